# Initial kernel scaffold; baseline (speedup 1.0000x reference)
import functools

import jax
import jax.numpy as jnp
from jax import lax
from jax.experimental import pallas as pl
from jax.experimental.pallas import tpu as pltpu
from jax.experimental.pallas import tpu_sc as plsc

NUM_CORES = 2
NUM_SUBCORES = 16
NUM_WORKERS = NUM_CORES * NUM_SUBCORES
CHUNK = 1024


def _gather_kernel(n_flat, d):
    b_per_w = n_flat // NUM_WORKERS
    n_chunks = b_per_w // CHUNK
    mesh = plsc.VectorSubcoreMesh(core_axis_name="c", subcore_axis_name="s")

    @functools.partial(
        pl.kernel,
        mesh=mesh,
        out_type=jax.ShapeDtypeStruct((n_flat, d), jnp.float32),
        scratch_types=[
            pltpu.VMEM((b_per_w,), jnp.int32),
            pltpu.VMEM((CHUNK, d), jnp.float32),
            pltpu.SemaphoreType.DMA,
        ],
    )
    def k(idx_hbm, table_hbm, out_hbm, idx_v, rows_v, sem):
        wid = lax.axis_index("s") * NUM_CORES + lax.axis_index("c")
        base = wid * b_per_w
        pltpu.sync_copy(idx_hbm.at[pl.ds(base, b_per_w)], idx_v)

        def body(c, carry):
            off = pl.multiple_of(c * CHUNK, CHUNK)
            pltpu.async_copy(
                table_hbm.at[idx_v.at[pl.ds(off, CHUNK)]], rows_v, sem
            ).wait()
            pltpu.sync_copy(rows_v, out_hbm.at[pl.ds(base + off, CHUNK)])
            return carry

        lax.fori_loop(0, n_chunks, body, 0)

    return k


def kernel(indices, table):
    batch, seq = indices.shape
    vocab, d = table.shape
    n_flat = batch * seq
    flat_idx = indices.reshape(n_flat).astype(jnp.int32)
    out = _gather_kernel(n_flat, d)(flat_idx, table)
    return out.reshape(batch, seq, d)


# speedup vs baseline: 2.7799x; 2.7799x over previous
import functools

import jax
import jax.numpy as jnp
from jax import lax
from jax.experimental import pallas as pl
from jax.experimental.pallas import tpu as pltpu
from jax.experimental.pallas import tpu_sc as plsc

NUM_CORES = 2
NUM_SUBCORES = 16
NUM_WORKERS = NUM_CORES * NUM_SUBCORES
CHUNK = 128


def _gather_kernel(n_flat, d_pad):
    b_per_w = n_flat // NUM_WORKERS
    n_chunks = b_per_w // CHUNK
    mesh = plsc.VectorSubcoreMesh(core_axis_name="c", subcore_axis_name="s")

    @functools.partial(
        pl.kernel,
        mesh=mesh,
        compiler_params=pltpu.CompilerParams(use_tc_tiling_on_sc=False),
        out_type=jax.ShapeDtypeStruct((n_flat, d_pad), jnp.float32),
        scratch_types=[
            pltpu.VMEM((CHUNK,), jnp.int32),
            pltpu.VMEM((CHUNK, d_pad), jnp.float32),
            pltpu.SemaphoreType.DMA,
        ],
    )
    def k(idx_hbm, table_hbm, out_hbm, idx_v, rows_v, sem):
        wid = lax.axis_index("s") * NUM_CORES + lax.axis_index("c")
        base = wid * b_per_w

        def body(c, carry):
            off = base + pl.multiple_of(c * CHUNK, CHUNK)
            pltpu.sync_copy(idx_hbm.at[pl.ds(off, CHUNK)], idx_v)
            pltpu.async_copy(table_hbm.at[idx_v], rows_v, sem).wait()
            pltpu.sync_copy(rows_v, out_hbm.at[pl.ds(off, CHUNK)])
            return carry

        lax.fori_loop(0, n_chunks, body, 0)

    return k


def kernel(indices, table):
    batch, seq = indices.shape
    vocab, d = table.shape
    d_pad = (d + 7) // 8 * 8
    n_flat = batch * seq
    flat_idx = indices.reshape(n_flat).astype(jnp.int32)
    table_pad = jnp.pad(table, ((0, 0), (0, d_pad - d)))
    out = _gather_kernel(n_flat, d_pad)(flat_idx, table_pad)
    return out[:, :d].reshape(batch, seq, d)


# speedup vs baseline: 3.6210x; 1.3025x over previous
import functools

import jax
import jax.numpy as jnp
from jax import lax
from jax.experimental import pallas as pl
from jax.experimental.pallas import tpu as pltpu
from jax.experimental.pallas import tpu_sc as plsc

NUM_CORES = 2
NUM_SUBCORES = 16
NUM_WORKERS = NUM_CORES * NUM_SUBCORES
CHUNK = 128
NBUF = 8


def _gather_kernel(n_flat, d, d_pad):
    b_per_w = n_flat // NUM_WORKERS
    n_chunks = b_per_w // CHUNK
    n_groups = n_chunks // NBUF - 1
    mesh = plsc.VectorSubcoreMesh(core_axis_name="c", subcore_axis_name="s")

    @functools.partial(
        pl.kernel,
        mesh=mesh,
        compiler_params=pltpu.CompilerParams(use_tc_tiling_on_sc=False),
        out_type=jax.ShapeDtypeStruct((n_flat, d_pad), jnp.float32),
        scratch_types=[
            pltpu.VMEM((n_chunks, CHUNK), jnp.int32),
            pltpu.VMEM((NBUF, CHUNK, d_pad), jnp.float32),
            pltpu.SemaphoreType.DMA((NBUF,)),
            pltpu.SemaphoreType.DMA((NBUF,)),
        ],
    )
    def k(idx_hbm, table_hbm, out_hbm, idx_v, rows_v, gsem, wsem):
        wid = lax.axis_index("s") * NUM_CORES + lax.axis_index("c")
        base = wid * b_per_w
        pltpu.sync_copy(idx_hbm.at[pl.ds(wid * n_chunks, n_chunks)], idx_v)

        def gather(c, b):
            pltpu.async_copy(
                table_hbm.at[idx_v.at[c]], rows_v.at[b], gsem.at[b]
            )

        def write(c, b):
            off = base + c * CHUNK
            pltpu.async_copy(
                rows_v.at[b], out_hbm.at[pl.ds(off, CHUNK)], wsem.at[b]
            )

        def wait_gather(b):
            pltpu.make_async_copy(
                table_hbm.at[pl.ds(0, CHUNK)], rows_v.at[b], gsem.at[b]
            ).wait()

        def wait_write(b):
            pltpu.make_async_copy(
                rows_v.at[b], out_hbm.at[pl.ds(0, CHUNK)], wsem.at[b]
            ).wait()

        for b in range(NBUF):
            gather(b, b)

        def body(g, carry):
            c0 = g * NBUF
            for b in range(NBUF):
                c = c0 + b
                wait_gather(b)
                write(c, b)
                wait_write(b)
                gather(c + NBUF, b)
            return carry

        lax.fori_loop(0, n_groups, body, 0)

        for b in range(NBUF):
            c = n_groups * NBUF + b
            wait_gather(b)
            write(c, b)
        for b in range(NBUF):
            wait_write(b)

    return k


def kernel(indices, table):
    batch, seq = indices.shape
    vocab, d = table.shape
    d_pad = (d + 7) // 8 * 8
    n_flat = batch * seq
    idx2d = indices.reshape(n_flat // CHUNK, CHUNK).astype(jnp.int32)
    table_pad = jnp.pad(table, ((0, 0), (0, d_pad - d)))
    out = _gather_kernel(n_flat, d, d_pad)(idx2d, table_pad)
    return out[:, :d].reshape(batch, seq, d)
